# baseline (speedup 1.0000x reference)
import functools

import jax
import jax.numpy as jnp
from jax import lax
from jax.experimental import pallas as pl
from jax.experimental.pallas import tpu as pltpu
from jax.experimental.pallas import tpu_sc as plsc

B = 16384
D = 128
H = 512
OUT = 3
OUT_PAD = 8

_info = plsc.get_sparse_core_info()
_NC, _NS = _info.num_cores, _info.num_subcores
_NW = _NC * _NS
_CH = (B // 128) // _NW


def _sc_gather(table, idx2d):
    mesh = plsc.VectorSubcoreMesh(core_axis_name="c", subcore_axis_name="s")

    @functools.partial(
        pl.kernel,
        mesh=mesh,
        out_type=jax.ShapeDtypeStruct((B, D), jnp.float32),
        scratch_types=[
            pltpu.VMEM((_CH, 128), jnp.int32),
            pltpu.VMEM((_CH, 128, D), jnp.float32),
            pltpu.SemaphoreType.DMA,
        ],
    )
    def k(table_hbm, idx_hbm, out_hbm, idx_v, rows_v, sem):
        wid = lax.axis_index("s") * _NC + lax.axis_index("c")
        base = wid * _CH
        pltpu.sync_copy(idx_hbm.at[pl.ds(base, _CH)], idx_v)
        copies = [
            pltpu.async_copy(table_hbm.at[idx_v.at[j]], rows_v.at[j], sem)
            for j in range(_CH)
        ]
        for c in copies:
            c.wait()
        for j in range(_CH):
            pltpu.sync_copy(
                rows_v.at[j], out_hbm.at[pl.ds((base + j) * 128, 128)]
            )

    return k(table, idx2d)


def _tc_mlp(x, w1t, b1r, w2t, b2r):
    BLK = 2048

    def body(x_ref, w1_ref, b1_ref, w2_ref, b2_ref, o_ref):
        h = jnp.dot(x_ref[:], w1_ref[:], preferred_element_type=jnp.float32)
        h = jnp.maximum(h + b1_ref[:], 0.0)
        o_ref[:] = (
            jnp.dot(h, w2_ref[:], preferred_element_type=jnp.float32) + b2_ref[:]
        )

    return pl.pallas_call(
        body,
        grid=(B // BLK,),
        in_specs=[
            pl.BlockSpec((BLK, D), lambda i: (i, 0)),
            pl.BlockSpec((D, H), lambda i: (0, 0)),
            pl.BlockSpec((1, H), lambda i: (0, 0)),
            pl.BlockSpec((H, OUT), lambda i: (0, 0)),
            pl.BlockSpec((1, OUT), lambda i: (0, 0)),
        ],
        out_specs=pl.BlockSpec((BLK, OUT), lambda i: (i, 0)),
        out_shape=jax.ShapeDtypeStruct((B, OUT), jnp.float32),
    )(x, w1t, b1r, w2t, b2r)


def kernel(geo_id, emb_table, W1, b1, W2, b2):
    idx2d = geo_id.astype(jnp.int32).reshape(B // 128, 128)
    x = _sc_gather(emb_table, idx2d)
    out = _tc_mlp(x, W1.T, b1.reshape(1, H), W2.T, b2.reshape(1, OUT))
    return out


# speedup vs baseline: 1.2211x; 1.2211x over previous
import functools

import jax
import jax.numpy as jnp
from jax import lax
from jax.experimental import pallas as pl
from jax.experimental.pallas import tpu as pltpu
from jax.experimental.pallas import tpu_sc as plsc

B = 16384
D = 128
H = 512
OUT = 3
OUT_PAD = 8

_info = plsc.get_sparse_core_info()
_NC, _NS = _info.num_cores, _info.num_subcores
_NW = _NC * _NS
_CH = (B // 128) // _NW


def _sc_gather(table, idx2d):
    mesh = plsc.VectorSubcoreMesh(core_axis_name="c", subcore_axis_name="s")

    @functools.partial(
        pl.kernel,
        mesh=mesh,
        out_type=jax.ShapeDtypeStruct((B, D), jnp.float32),
        scratch_types=[
            pltpu.VMEM((_CH, 128), jnp.int32),
            pltpu.VMEM((_CH, 128, D), jnp.float32),
            pltpu.SemaphoreType.DMA,
        ],
    )
    def k(table_hbm, idx_hbm, out_hbm, idx_v, rows_v, sem):
        wid = lax.axis_index("s") * _NC + lax.axis_index("c")
        base = wid * _CH
        pltpu.sync_copy(idx_hbm.at[pl.ds(base, _CH)], idx_v)
        copies = [
            pltpu.async_copy(table_hbm.at[idx_v.at[j]], rows_v.at[j], sem)
            for j in range(_CH)
        ]
        for c in copies:
            c.wait()
        for j in range(_CH):
            pltpu.sync_copy(
                rows_v.at[j], out_hbm.at[pl.ds((base + j) * 128, 128)]
            )

    return k(table, idx2d)


def _tc_mlp(x, w1t, b1r, w2t, b2r):
    BLK = 2048

    def body(x_ref, w1_ref, b1_ref, w2_ref, b2_ref, o_ref):
        h = jnp.dot(x_ref[:], w1_ref[:], preferred_element_type=jnp.float32)
        h = jnp.maximum(h + b1_ref[:], 0.0)
        ot = lax.dot_general(
            w2_ref[:], h, (((1,), (1,)), ((), ())),
            preferred_element_type=jnp.float32,
        )
        o_ref[:] = ot + b2_ref[:]

    return pl.pallas_call(
        body,
        grid=(B // BLK,),
        in_specs=[
            pl.BlockSpec((BLK, D), lambda i: (i, 0)),
            pl.BlockSpec((D, H), lambda i: (0, 0)),
            pl.BlockSpec((1, H), lambda i: (0, 0)),
            pl.BlockSpec((OUT, H), lambda i: (0, 0)),
            pl.BlockSpec((OUT, 1), lambda i: (0, 0)),
        ],
        out_specs=pl.BlockSpec((OUT, BLK), lambda i: (0, i)),
        out_shape=jax.ShapeDtypeStruct((OUT, B), jnp.float32),
    )(x, w1t, b1r, w2t, b2r)


def kernel(geo_id, emb_table, W1, b1, W2, b2):
    idx2d = geo_id.astype(jnp.int32).reshape(B // 128, 128)
    x = _sc_gather(emb_table, idx2d)
    out_t = _tc_mlp(x, W1.T, b1.reshape(1, H), W2, b2.reshape(OUT, 1))
    return out_t.T
